# baseline (speedup 1.0000x reference)
import jax
import jax.numpy as jnp
from jax.experimental import pallas as pl

_N = 10000
_D = 128
_TILE_M = 400


def _support_body(f_ref, w_ref, out_ref):
    out_ref[...] = jnp.dot(
        f_ref[...], w_ref[...], preferred_element_type=jnp.float32
    ).astype(jnp.bfloat16)


def _spmm_body(adj_ref, sup_ref, b_ref, out_ref):
    a16 = adj_ref[...].astype(jnp.bfloat16)
    acc = jnp.dot(a16, sup_ref[...], preferred_element_type=jnp.float32)
    out_ref[...] = acc + b_ref[...]


def kernel(feature, modality_number, adjencency_matrix, W, b):
    feature_f32 = feature.astype(jnp.float32)

    support = pl.pallas_call(
        _support_body,
        out_shape=jax.ShapeDtypeStruct((_N, _D), jnp.bfloat16),
    )(feature_f32, W)

    gcn = pl.pallas_call(
        _spmm_body,
        grid=(_N // _TILE_M,),
        in_specs=[
            pl.BlockSpec((_TILE_M, _N), lambda i: (i, 0)),
            pl.BlockSpec((_N, _D), lambda i: (0, 0)),
            pl.BlockSpec((1, _D), lambda i: (0, 0)),
        ],
        out_specs=pl.BlockSpec((_TILE_M, _D), lambda i: (i, 0)),
        out_shape=jax.ShapeDtypeStruct((_N, _D), jnp.float32),
    )(adjencency_matrix, support, b.reshape(1, _D))

    return jnp.where(modality_number > 1, gcn, feature_f32)


# speedup vs baseline: 1.0004x; 1.0004x over previous
import jax
import jax.numpy as jnp
from jax.experimental import pallas as pl
from jax.experimental.pallas import tpu as pltpu

_N = 10000
_D = 128
_TILE_M = 400


def _gcn_body(adj_ref, f_ref, w_ref, b_ref, out_ref, sup_ref):
    i = pl.program_id(0)

    @pl.when(i == 0)
    def _():
        sup_ref[...] = jnp.dot(
            f_ref[...].astype(jnp.bfloat16),
            w_ref[...].astype(jnp.bfloat16),
            preferred_element_type=jnp.float32,
        ).astype(jnp.bfloat16)

    acc = jnp.dot(
        adj_ref[...].astype(jnp.bfloat16),
        sup_ref[...],
        preferred_element_type=jnp.float32,
    )
    out_ref[...] = acc + b_ref[...]


def kernel(feature, modality_number, adjencency_matrix, W, b):
    feature_f32 = feature.astype(jnp.float32)

    def gcn_branch(_):
        return pl.pallas_call(
            _gcn_body,
            grid=(_N // _TILE_M,),
            in_specs=[
                pl.BlockSpec((_TILE_M, _N), lambda i: (i, 0)),
                pl.BlockSpec((_N, _D), lambda i: (0, 0)),
                pl.BlockSpec((_D, _D), lambda i: (0, 0)),
                pl.BlockSpec((1, _D), lambda i: (0, 0)),
            ],
            out_specs=pl.BlockSpec((_TILE_M, _D), lambda i: (i, 0)),
            out_shape=jax.ShapeDtypeStruct((_N, _D), jnp.float32),
            scratch_shapes=[pltpu.VMEM((_N, _D), jnp.bfloat16)],
        )(adjencency_matrix, feature_f32, W, b.reshape(1, _D))

    return jax.lax.cond(modality_number > 1, gcn_branch, lambda _: feature_f32, None)


# speedup vs baseline: 1.0110x; 1.0106x over previous
import jax
import jax.numpy as jnp
from jax.experimental import pallas as pl
from jax.experimental.pallas import tpu as pltpu

_N = 10000
_D = 128
_TILE_M = 400


def _gcn_body(adj_ref, f_ref, w_ref, b_ref, out_ref, sup_ref):
    i = pl.program_id(0)

    @pl.when(i == 0)
    def _():
        sup_ref[...] = jnp.dot(
            f_ref[...],
            w_ref[...],
            preferred_element_type=jnp.float32,
        )

    acc = jnp.dot(
        adj_ref[...],
        sup_ref[...],
        precision=jax.lax.Precision.DEFAULT,
        preferred_element_type=jnp.float32,
    )
    out_ref[...] = acc + b_ref[...]


def kernel(feature, modality_number, adjencency_matrix, W, b):
    feature_f32 = feature.astype(jnp.float32)

    def gcn_branch(_):
        return pl.pallas_call(
            _gcn_body,
            grid=(_N // _TILE_M,),
            in_specs=[
                pl.BlockSpec((_TILE_M, _N), lambda i: (i, 0)),
                pl.BlockSpec((_N, _D), lambda i: (0, 0)),
                pl.BlockSpec((_D, _D), lambda i: (0, 0)),
                pl.BlockSpec((1, _D), lambda i: (0, 0)),
            ],
            out_specs=pl.BlockSpec((_TILE_M, _D), lambda i: (i, 0)),
            out_shape=jax.ShapeDtypeStruct((_N, _D), jnp.float32),
            scratch_shapes=[pltpu.VMEM((_N, _D), jnp.float32)],
        )(adjencency_matrix, feature_f32, W, b.reshape(1, _D))

    return jax.lax.cond(modality_number > 1, gcn_branch, lambda _: feature_f32, None)
